# Initial kernel scaffold; baseline (speedup 1.0000x reference)
import jax
import jax.numpy as jnp
from jax.experimental import pallas as pl


def kernel(h, x, edge_index, params):
    raise NotImplementedError("write your pallas kernel here")


# speedup vs baseline: 3.3302x; 3.3302x over previous
import functools

import jax
import jax.numpy as jnp
from jax import lax
from jax.experimental import pallas as pl
from jax.experimental.pallas import tpu as pltpu
from jax.experimental.pallas import tpu_sc as plsc

D = 48
NW = 32
KE = 80


@functools.cache
def _make_gather(N, E):
    per_w = E // NW
    B = per_w // KE
    mesh = plsc.VectorSubcoreMesh(core_axis_name="c", subcore_axis_name="s")

    @functools.partial(
        pl.kernel,
        out_type=[jax.ShapeDtypeStruct((E, D), jnp.float32),
                  jax.ShapeDtypeStruct((E, D), jnp.float32)],
        mesh=mesh,
        compiler_params=pltpu.CompilerParams(use_tc_tiling_on_sc=False),
        scratch_types=[pltpu.VMEM((B, KE), jnp.int32),
                       pltpu.VMEM((B, KE), jnp.int32),
                       pltpu.VMEM((KE, D), jnp.float32),
                       pltpu.VMEM((KE, D), jnp.float32),
                       pltpu.SemaphoreType.DMA,
                       pltpu.SemaphoreType.DMA],
    )
    def gather(tab, dst3, src3, rd_out, rs_out,
               idx_d, idx_s, buf_d, buf_s, sem_d, sem_s):
        c = lax.axis_index("c")
        s = lax.axis_index("s")
        w = s * 2 + c
        pltpu.sync_copy(dst3.at[w], idx_d)
        pltpu.sync_copy(src3.at[w], idx_s)

        def body(i, carry):
            off = pl.multiple_of((w * B + i) * KE, KE)
            cp_d = pltpu.async_copy(tab.at[idx_d.at[i]], buf_d, sem_d)
            cp_s = pltpu.async_copy(tab.at[idx_s.at[i]], buf_s, sem_s)
            cp_d.wait()
            pltpu.sync_copy(buf_d, rd_out.at[pl.ds(off, KE)])
            cp_s.wait()
            pltpu.sync_copy(buf_s, rs_out.at[pl.ds(off, KE)])
            return carry

        lax.fori_loop(0, B, body, 0)

    return gather


@functools.cache
def _make_scatter(N, E):
    per_w = E // NW
    B = per_w // KE
    stripe = (N // 16) & ~7
    tail = N - 16 * stripe
    mesh = plsc.VectorSubcoreMesh(core_axis_name="c", subcore_axis_name="s")

    @functools.partial(
        pl.kernel,
        out_type=jax.ShapeDtypeStruct((2, N, D), jnp.float32),
        mesh=mesh,
        compiler_params=pltpu.CompilerParams(use_tc_tiling_on_sc=False),
        scratch_types=[pltpu.VMEM_SHARED((N, D), jnp.float32),
                       pltpu.VMEM((B, KE), jnp.int32),
                       pltpu.VMEM((KE, D), jnp.float32)],
    )
    def scatter(msg, dst3, zeros, out, shared, idx_d, buf):
        c = lax.axis_index("c")
        s = lax.axis_index("s")
        w = s * 2 + c
        soff = pl.multiple_of(s * stripe, 8)
        pltpu.sync_copy(zeros.at[pl.ds(soff, stripe)],
                        shared.at[pl.ds(soff, stripe)])
        if tail:
            @pl.when(s == 0)
            def _():
                pltpu.sync_copy(zeros.at[pl.ds(16 * stripe, tail)],
                                shared.at[pl.ds(16 * stripe, tail)])
        pltpu.sync_copy(dst3.at[w], idx_d)
        plsc.subcore_barrier()

        def body(i, carry):
            off = pl.multiple_of((w * B + i) * KE, KE)
            pltpu.sync_copy(msg.at[pl.ds(off, KE)], buf)
            pltpu.sync_copy(buf, shared.at[idx_d.at[i]], add=True)
            return carry

        lax.fori_loop(0, B, body, 0)
        plsc.subcore_barrier()
        pltpu.sync_copy(shared.at[pl.ds(soff, stripe)],
                        out.at[c, pl.ds(soff, stripe)])
        if tail:
            @pl.when(s == 0)
            def _():
                pltpu.sync_copy(shared.at[pl.ds(16 * stripe, tail)],
                                out.at[c, pl.ds(16 * stripe, tail)])

    return scatter


def _leaky(v):
    return jnp.where(v > 0, v, 0.01 * v)


def _rsqrt(v):
    r = jax.lax.rsqrt(v)
    return r * (1.5 - 0.5 * v * r * r)


def _sqrt0(v):
    return jnp.where(v > 0, v * _rsqrt(jnp.maximum(v, 1e-30)), 0.0)


def _embed_body(h_ref, w_ref, b_ref, g_ref, bb_ref, o_ref):
    y = jnp.dot(h_ref[...], w_ref[...], preferred_element_type=jnp.float32)
    y = y + b_ref[...]
    mu = jnp.mean(y, axis=0, keepdims=True)
    var = jnp.mean(y * y, axis=0, keepdims=True) - mu * mu
    o_ref[...] = _leaky((y - mu) * _rsqrt(var + 1e-5) * g_ref[...]
                        + bb_ref[...])


def _stats_body(rd_ref, rs_ref, srd_ref, qrd_ref, srs_ref, qrs_ref, dst_ref):
    i = pl.program_id(0)

    @pl.when(i == 0)
    def _():
        srd_ref[...] = jnp.zeros_like(srd_ref)
        qrd_ref[...] = jnp.zeros_like(qrd_ref)
        srs_ref[...] = jnp.zeros_like(srs_ref)
        qrs_ref[...] = jnp.zeros_like(qrs_ref)
        dst_ref[...] = jnp.zeros_like(dst_ref)

    rd = rd_ref[...]
    rs = rs_ref[...]
    diffp = rd[:, 32:48] - rs[:, 32:48]
    dd = jnp.sum(diffp * diffp, axis=1, keepdims=True)
    d = _sqrt0(dd)
    srd_ref[...] += jnp.sum(rd, axis=0, keepdims=True)
    qrd_ref[...] += jnp.sum(rd * rd, axis=0, keepdims=True)
    srs_ref[...] += jnp.sum(rs, axis=0, keepdims=True)
    qrs_ref[...] += jnp.sum(rs * rs, axis=0, keepdims=True)
    dst_ref[...] += jnp.concatenate(
        [jnp.full((1, 8), jnp.sum(d)), jnp.full((1, 8), jnp.sum(dd))], axis=0)


def _edge_body(rd_ref, rs_ref, si_ref, ti_ref, sj_ref, tj_ref, sd_ref,
               td_ref, w1a_ref, w1b_ref, w1d_ref, b1_ref,
               w2_ref, b2_ref, cw1_ref, cb1_ref, cw2_ref, o_ref):
    f32 = jnp.float32
    rd = rd_ref[...]
    rs = rs_ref[...]
    hi_bn = rd[:, :32] * si_ref[...] + ti_ref[...]
    hj_bn = rs[:, :32] * sj_ref[...] + tj_ref[...]
    diffp = rd[:, 32:48] - rs[:, 32:48]
    d = _sqrt0(jnp.sum(diffp * diffp, axis=1, keepdims=True))
    d_bn = d * sd_ref[...] + td_ref[...]
    dq = d_bn.astype(jnp.bfloat16).astype(f32)
    wq = w1d_ref[...].astype(jnp.bfloat16).astype(f32)
    m = jnp.dot(hi_bn, w1a_ref[...], preferred_element_type=f32)
    m = m + jnp.dot(hj_bn, w1b_ref[...], preferred_element_type=f32)
    m = _leaky(m + dq * wq + b1_ref[...])
    m = _leaky(jnp.dot(m, w2_ref[...], preferred_element_type=f32)
               + b2_ref[...])
    cc = _leaky(jnp.dot(m, cw1_ref[...], preferred_element_type=f32)
                + cb1_ref[...])
    c3 = jnp.dot(cc, cw2_ref[...], preferred_element_type=f32)
    o_ref[...] = jnp.concatenate([m, diffp * c3], axis=1)


def _node_body(p_ref, h_ref, xp_ref, w1a_ref, w1b_ref, b1_ref, g_ref,
               bb_ref, w2_ref, b2_ref, ho_ref, xo_ref):
    agg = p_ref[0] + p_ref[1]
    m_agg = agg[:, :32]
    xo_ref[...] = xp_ref[...] + agg[:, 32:48]
    h = h_ref[...]
    z = jnp.dot(h, w1a_ref[...], preferred_element_type=jnp.float32)
    z = z + jnp.dot(m_agg, w1b_ref[...], preferred_element_type=jnp.float32)
    z = z + b1_ref[...]
    mu = jnp.mean(z, axis=0, keepdims=True)
    var = jnp.mean(z * z, axis=0, keepdims=True) - mu * mu
    z = _leaky((z - mu) * _rsqrt(var + 1e-5) * g_ref[...] + bb_ref[...])
    ho_ref[...] = jnp.dot(z, w2_ref[...], preferred_element_type=jnp.float32)
    ho_ref[...] += b2_ref[...]


def _out_body(h_ref, w_ref, b_ref, o_ref):
    o_ref[...] = jnp.dot(h_ref[...], w_ref[...],
                         preferred_element_type=jnp.float32) + b_ref[...]


def kernel(h, x, edge_index, params):
    N, IN = h.shape
    E = edge_index.shape[1]
    f32 = jnp.float32

    B = E // NW // KE
    src3 = edge_index[0].astype(jnp.int32).reshape(NW, B, KE)
    dst3 = edge_index[1].astype(jnp.int32).reshape(NW, B, KE)
    xp = jnp.pad(x, ((0, 0), (0, 16 - x.shape[1])))

    r2 = lambda v: v[None, :]

    tc_params = pltpu.CompilerParams(vmem_limit_bytes=100 * 1024 * 1024)
    hh = pl.pallas_call(
        _embed_body,
        compiler_params=tc_params,
        out_shape=jax.ShapeDtypeStruct((N, 32), f32),
    )(h, params['ri_W'].T, r2(params['ri_b']), r2(params['ri_g']),
      r2(params['ri_bb']))

    gather = _make_gather(N, E)
    scatter = _make_scatter(N, E)
    zeros48 = jnp.zeros((N, D), f32)

    KB = 4000
    grid_e = E // KB
    blk = lambda shape: pl.BlockSpec(shape, lambda i: (0, 0))
    eblk = pl.BlockSpec((KB, D), lambda i: (i, 0))

    for l in range(2):
        p = lambda k: params['l%d_' % l + k]
        tab = jnp.concatenate([hh, xp], axis=1)
        rd, rs = gather(tab, dst3, src3)

        srd, qrd, srs, qrs, dsums = pl.pallas_call(
            _stats_body,
            grid=(grid_e,),
            in_specs=[eblk, eblk],
            out_specs=[blk((1, D)), blk((1, D)), blk((1, D)), blk((1, D)),
                       blk((2, 8))],
            out_shape=[jax.ShapeDtypeStruct((1, D), f32)] * 4
            + [jax.ShapeDtypeStruct((2, 8), f32)],
        )(rd, rs)

        mu65 = jnp.concatenate([srd[0, :32], srs[0, :32], dsums[0, :1]]) / E
        ex2 = jnp.concatenate([qrd[0, :32], qrs[0, :32], dsums[1, :1]]) / E
        var65 = ex2 - mu65 * mu65
        s65 = p('ein_g') / jnp.sqrt(var65 + 1e-5)
        t65 = p('ein_b') - mu65 * s65
        W1 = p('e_W1')
        cw2p = jnp.pad(p('c_W2'), ((0, 13), (0, 0))).T

        msg = pl.pallas_call(
            _edge_body,
            grid=(grid_e,),
            in_specs=[eblk, eblk,
                      blk((1, 32)), blk((1, 32)), blk((1, 32)), blk((1, 32)),
                      blk((1, 1)), blk((1, 1)),
                      blk((32, 32)), blk((32, 32)), blk((1, 32)), blk((1, 32)),
                      blk((32, 32)), blk((1, 32)),
                      blk((32, 32)), blk((1, 32)), blk((32, 16))],
            out_specs=eblk,
            out_shape=jax.ShapeDtypeStruct((E, D), f32),
        )(rd, rs, r2(s65[:32]), r2(t65[:32]), r2(s65[32:64]), r2(t65[32:64]),
          s65[64:].reshape(1, 1), t65[64:].reshape(1, 1),
          W1[:, :32].T, W1[:, 32:64].T, W1[:, 64:65].T, r2(p('e_b1')),
          p('e_W2').T, r2(p('e_b2')), p('c_W1').T, r2(p('c_b1')), cw2p)

        partials = scatter(msg, dst3, zeros48)

        hh, xp = pl.pallas_call(
            _node_body,
            compiler_params=tc_params,
            out_shape=[jax.ShapeDtypeStruct((N, 32), f32),
                       jax.ShapeDtypeStruct((N, 16), f32)],
        )(partials, hh, xp, p('n_W1')[:, :32].T, p('n_W1')[:, 32:].T,
          r2(p('n_b1')), r2(p('n_g')), r2(p('n_bb')), p('n_W2').T,
          r2(p('n_b2')))

    out = pl.pallas_call(
        _out_body,
        compiler_params=tc_params,
        out_shape=jax.ShapeDtypeStruct((N, IN), f32),
    )(hh, params['ro_W'].T, r2(params['ro_b']))

    return jnp.concatenate([out, xp[:, :3]], axis=1)


# speedup vs baseline: 4.8261x; 1.4492x over previous
import functools

import jax
import jax.numpy as jnp
from jax import lax
from jax.experimental import pallas as pl
from jax.experimental.pallas import tpu as pltpu
from jax.experimental.pallas import tpu_sc as plsc

D = 48
NW = 32
KE = 80


@functools.cache
def _make_gather(N, E):
    per_w = E // NW
    B = per_w // KE
    mesh = plsc.VectorSubcoreMesh(core_axis_name="c", subcore_axis_name="s")

    @functools.partial(
        pl.kernel,
        out_type=[jax.ShapeDtypeStruct((E, 128), jnp.float32),
                  jax.ShapeDtypeStruct((E, 128), jnp.float32)],
        mesh=mesh,
        compiler_params=pltpu.CompilerParams(use_tc_tiling_on_sc=False),
        scratch_types=[pltpu.VMEM((B, KE), jnp.int32),
                       pltpu.VMEM((B, KE), jnp.int32),
                       pltpu.VMEM((KE, D), jnp.float32),
                       pltpu.VMEM((KE, D), jnp.float32),
                       pltpu.SemaphoreType.DMA,
                       pltpu.SemaphoreType.DMA],
    )
    def gather(tab, dst3, src3, rd_out, rs_out,
               idx_d, idx_s, buf_d, buf_s, sem_d, sem_s):
        c = lax.axis_index("c")
        s = lax.axis_index("s")
        w = s * 2 + c
        pltpu.sync_copy(dst3.at[w], idx_d)
        pltpu.sync_copy(src3.at[w], idx_s)

        def body(i, carry):
            off = pl.multiple_of((w * B + i) * KE, KE)
            cp_d = pltpu.async_copy(tab.at[idx_d.at[i]], buf_d, sem_d)
            cp_s = pltpu.async_copy(tab.at[idx_s.at[i]], buf_s, sem_s)
            cp_d.wait()
            pltpu.sync_copy(buf_d, rd_out.at[pl.ds(off, KE), pl.ds(0, D)])
            cp_s.wait()
            pltpu.sync_copy(buf_s, rs_out.at[pl.ds(off, KE), pl.ds(0, D)])
            return carry

        lax.fori_loop(0, B, body, 0)

    return gather


@functools.cache
def _make_scatter(N, E):
    per_w = E // NW
    B = per_w // KE
    stripe = (N // 16) & ~7
    tail = N - 16 * stripe
    mesh = plsc.VectorSubcoreMesh(core_axis_name="c", subcore_axis_name="s")

    @functools.partial(
        pl.kernel,
        out_type=jax.ShapeDtypeStruct((2, N, D), jnp.float32),
        mesh=mesh,
        compiler_params=pltpu.CompilerParams(use_tc_tiling_on_sc=False),
        scratch_types=[pltpu.VMEM_SHARED((N, D), jnp.float32),
                       pltpu.VMEM((B, KE), jnp.int32),
                       pltpu.VMEM((KE, D), jnp.float32)],
    )
    def scatter(msg, dst3, zeros, out, shared, idx_d, buf):
        c = lax.axis_index("c")
        s = lax.axis_index("s")
        w = s * 2 + c
        soff = pl.multiple_of(s * stripe, 8)
        pltpu.sync_copy(zeros.at[pl.ds(soff, stripe)],
                        shared.at[pl.ds(soff, stripe)])
        if tail:
            @pl.when(s == 0)
            def _():
                pltpu.sync_copy(zeros.at[pl.ds(16 * stripe, tail)],
                                shared.at[pl.ds(16 * stripe, tail)])
        pltpu.sync_copy(dst3.at[w], idx_d)
        plsc.subcore_barrier()

        def body(i, carry):
            off = pl.multiple_of((w * B + i) * KE, KE)
            pltpu.sync_copy(msg.at[pl.ds(off, KE), pl.ds(0, D)], buf)
            pltpu.sync_copy(buf, shared.at[idx_d.at[i]], add=True)
            return carry

        lax.fori_loop(0, B, body, 0)
        plsc.subcore_barrier()
        pltpu.sync_copy(shared.at[pl.ds(soff, stripe)],
                        out.at[c, pl.ds(soff, stripe)])
        if tail:
            @pl.when(s == 0)
            def _():
                pltpu.sync_copy(shared.at[pl.ds(16 * stripe, tail)],
                                out.at[c, pl.ds(16 * stripe, tail)])

    return scatter


def _leaky(v):
    return jnp.where(v > 0, v, 0.01 * v)


def _rsqrt(v):
    r = jax.lax.rsqrt(v)
    return r * (1.5 - 0.5 * v * r * r)


def _sqrt0(v):
    return jnp.where(v > 0, v * _rsqrt(jnp.maximum(v, 1e-30)), 0.0)


def _embed_body(h_ref, w_ref, b_ref, g_ref, bb_ref, o_ref):
    y = jnp.dot(h_ref[...], w_ref[...], preferred_element_type=jnp.float32)
    y = y + b_ref[...]
    mu = jnp.mean(y, axis=0, keepdims=True)
    var = jnp.mean(y * y, axis=0, keepdims=True) - mu * mu
    o_ref[...] = _leaky((y - mu) * _rsqrt(var + 1e-5) * g_ref[...]
                        + bb_ref[...])


def _stats_body(rd_ref, rs_ref, srd_ref, qrd_ref, srs_ref, qrs_ref, dst_ref):
    i = pl.program_id(0)

    @pl.when(i == 0)
    def _():
        srd_ref[...] = jnp.zeros_like(srd_ref)
        qrd_ref[...] = jnp.zeros_like(qrd_ref)
        srs_ref[...] = jnp.zeros_like(srs_ref)
        qrs_ref[...] = jnp.zeros_like(qrs_ref)
        dst_ref[...] = jnp.zeros_like(dst_ref)

    rd = rd_ref[...]
    rs = rs_ref[...]
    diffp = rd[:, 32:48] - rs[:, 32:48]
    dd = jnp.sum(diffp * diffp, axis=1, keepdims=True)
    d = _sqrt0(dd)
    srd_ref[...] += jnp.sum(rd, axis=0, keepdims=True)
    qrd_ref[...] += jnp.sum(rd * rd, axis=0, keepdims=True)
    srs_ref[...] += jnp.sum(rs, axis=0, keepdims=True)
    qrs_ref[...] += jnp.sum(rs * rs, axis=0, keepdims=True)
    dst_ref[...] += jnp.concatenate(
        [jnp.full((1, 8), jnp.sum(d)), jnp.full((1, 8), jnp.sum(dd))], axis=0)


def _edge_body(rd_ref, rs_ref, si_ref, ti_ref, sj_ref, tj_ref, sd_ref,
               td_ref, w1a_ref, w1b_ref, w1d_ref, b1_ref,
               w2_ref, b2_ref, cw1_ref, cb1_ref, cw2_ref, o_ref):
    f32 = jnp.float32
    rd = rd_ref[...]
    rs = rs_ref[...]
    hi_bn = rd[:, :32] * si_ref[...] + ti_ref[...]
    hj_bn = rs[:, :32] * sj_ref[...] + tj_ref[...]
    diffp = rd[:, 32:48] - rs[:, 32:48]
    d = _sqrt0(jnp.sum(diffp * diffp, axis=1, keepdims=True))
    d_bn = d * sd_ref[...] + td_ref[...]
    dq = d_bn.astype(jnp.bfloat16).astype(f32)
    wq = w1d_ref[...].astype(jnp.bfloat16).astype(f32)
    m = jnp.dot(hi_bn, w1a_ref[...], preferred_element_type=f32)
    m = m + jnp.dot(hj_bn, w1b_ref[...], preferred_element_type=f32)
    m = _leaky(m + dq * wq + b1_ref[...])
    m = _leaky(jnp.dot(m, w2_ref[...], preferred_element_type=f32)
               + b2_ref[...])
    cc = _leaky(jnp.dot(m, cw1_ref[...], preferred_element_type=f32)
                + cb1_ref[...])
    c3 = jnp.dot(cc, cw2_ref[...], preferred_element_type=f32)
    o_ref[...] = jnp.concatenate(
        [m, diffp * c3, jnp.zeros((m.shape[0], 80), jnp.float32)], axis=1)


def _node_body(p_ref, h_ref, xp_ref, w1a_ref, w1b_ref, b1_ref, g_ref,
               bb_ref, w2_ref, b2_ref, ho_ref, xo_ref):
    agg = p_ref[0] + p_ref[1]
    m_agg = agg[:, :32]
    xo_ref[...] = xp_ref[...] + agg[:, 32:48]
    h = h_ref[...]
    z = jnp.dot(h, w1a_ref[...], preferred_element_type=jnp.float32)
    z = z + jnp.dot(m_agg, w1b_ref[...], preferred_element_type=jnp.float32)
    z = z + b1_ref[...]
    mu = jnp.mean(z, axis=0, keepdims=True)
    var = jnp.mean(z * z, axis=0, keepdims=True) - mu * mu
    z = _leaky((z - mu) * _rsqrt(var + 1e-5) * g_ref[...] + bb_ref[...])
    ho_ref[...] = jnp.dot(z, w2_ref[...], preferred_element_type=jnp.float32)
    ho_ref[...] += b2_ref[...]


def _out_body(h_ref, w_ref, b_ref, o_ref):
    o_ref[...] = jnp.dot(h_ref[...], w_ref[...],
                         preferred_element_type=jnp.float32) + b_ref[...]


def kernel(h, x, edge_index, params):
    N, IN = h.shape
    E = edge_index.shape[1]
    f32 = jnp.float32

    B = E // NW // KE
    src3 = edge_index[0].astype(jnp.int32).reshape(NW, B, KE)
    dst3 = edge_index[1].astype(jnp.int32).reshape(NW, B, KE)
    xp = jnp.pad(x, ((0, 0), (0, 16 - x.shape[1])))

    r2 = lambda v: v[None, :]

    tc_params = pltpu.CompilerParams(vmem_limit_bytes=100 * 1024 * 1024)
    hh = pl.pallas_call(
        _embed_body,
        compiler_params=tc_params,
        out_shape=jax.ShapeDtypeStruct((N, 32), f32),
    )(h, params['ri_W'].T, r2(params['ri_b']), r2(params['ri_g']),
      r2(params['ri_bb']))

    gather = _make_gather(N, E)
    scatter = _make_scatter(N, E)
    zeros48 = jnp.zeros((N, D), f32)

    KB = 4000
    grid_e = E // KB
    blk = lambda shape: pl.BlockSpec(shape, lambda i: (0, 0))
    eblk = pl.BlockSpec((KB, 128), lambda i: (i, 0))

    for l in range(2):
        p = lambda k: params['l%d_' % l + k]
        tab = jnp.concatenate([hh, xp], axis=1)
        rd, rs = gather(tab, dst3, src3)

        srd, qrd, srs, qrs, dsums = pl.pallas_call(
            _stats_body,
            grid=(grid_e,),
            in_specs=[eblk, eblk],
            out_specs=[blk((1, 128)), blk((1, 128)), blk((1, 128)),
                       blk((1, 128)), blk((2, 8))],
            out_shape=[jax.ShapeDtypeStruct((1, 128), f32)] * 4
            + [jax.ShapeDtypeStruct((2, 8), f32)],
        )(rd, rs)

        mu65 = jnp.concatenate([srd[0, :32], srs[0, :32], dsums[0, :1]]) / E
        ex2 = jnp.concatenate([qrd[0, :32], qrs[0, :32], dsums[1, :1]]) / E
        var65 = ex2 - mu65 * mu65
        s65 = p('ein_g') / jnp.sqrt(var65 + 1e-5)
        t65 = p('ein_b') - mu65 * s65
        W1 = p('e_W1')
        cw2p = jnp.pad(p('c_W2'), ((0, 13), (0, 0))).T

        msg = pl.pallas_call(
            _edge_body,
            grid=(grid_e,),
            in_specs=[eblk, eblk,
                      blk((1, 32)), blk((1, 32)), blk((1, 32)), blk((1, 32)),
                      blk((1, 1)), blk((1, 1)),
                      blk((32, 32)), blk((32, 32)), blk((1, 32)), blk((1, 32)),
                      blk((32, 32)), blk((1, 32)),
                      blk((32, 32)), blk((1, 32)), blk((32, 16))],
            out_specs=eblk,
            out_shape=jax.ShapeDtypeStruct((E, 128), f32),
        )(rd, rs, r2(s65[:32]), r2(t65[:32]), r2(s65[32:64]), r2(t65[32:64]),
          s65[64:].reshape(1, 1), t65[64:].reshape(1, 1),
          W1[:, :32].T, W1[:, 32:64].T, W1[:, 64:65].T, r2(p('e_b1')),
          p('e_W2').T, r2(p('e_b2')), p('c_W1').T, r2(p('c_b1')), cw2p)

        partials = scatter(msg, dst3, zeros48)

        hh, xp = pl.pallas_call(
            _node_body,
            compiler_params=tc_params,
            out_shape=[jax.ShapeDtypeStruct((N, 32), f32),
                       jax.ShapeDtypeStruct((N, 16), f32)],
        )(partials, hh, xp, p('n_W1')[:, :32].T, p('n_W1')[:, 32:].T,
          r2(p('n_b1')), r2(p('n_g')), r2(p('n_bb')), p('n_W2').T,
          r2(p('n_b2')))

    out = pl.pallas_call(
        _out_body,
        compiler_params=tc_params,
        out_shape=jax.ShapeDtypeStruct((N, IN), f32),
    )(hh, params['ro_W'].T, r2(params['ro_b']))

    return jnp.concatenate([out, xp[:, :3]], axis=1)


# speedup vs baseline: 5.1852x; 1.0744x over previous
import functools

import jax
import jax.numpy as jnp
from jax import lax
from jax.experimental import pallas as pl
from jax.experimental.pallas import tpu as pltpu
from jax.experimental.pallas import tpu_sc as plsc

D = 48
NW = 32
KE = 80


@functools.cache
def _make_gather(N, E):
    per_w = E // NW
    B = per_w // KE
    mesh = plsc.VectorSubcoreMesh(core_axis_name="c", subcore_axis_name="s")

    @functools.partial(
        pl.kernel,
        out_type=[jax.ShapeDtypeStruct((E, 128), jnp.float32),
                  jax.ShapeDtypeStruct((E, 128), jnp.float32)],
        mesh=mesh,
        compiler_params=pltpu.CompilerParams(use_tc_tiling_on_sc=False),
        scratch_types=[pltpu.VMEM((B, KE), jnp.int32),
                       pltpu.VMEM((B, KE), jnp.int32)]
        + [pltpu.VMEM((KE, D), jnp.float32)] * 4
        + [pltpu.SemaphoreType.DMA] * 8,
    )
    def gather(tab, dst3, src3, rd_out, rs_out,
               idx_d, idx_s, bd0, bs0, bd1, bs1,
               gd0, gs0, gd1, gs1, wd0, ws0, wd1, ws1):
        c = lax.axis_index("c")
        s = lax.axis_index("s")
        w = s * 2 + c
        pltpu.sync_copy(dst3.at[w], idx_d)
        pltpu.sync_copy(src3.at[w], idx_s)

        sets = ((bd0, bs0, gd0, gs0, wd0, ws0),
                (bd1, bs1, gd1, gs1, wd1, ws1))

        def g_start(i, p):
            bd, bs, gd, gs, _, _ = sets[p]
            pltpu.async_copy(tab.at[idx_d.at[i]], bd, gd)
            pltpu.async_copy(tab.at[idx_s.at[i]], bs, gs)

        def g_wait(p):
            bd, bs, gd, gs, _, _ = sets[p]
            pltpu.make_async_copy(tab.at[idx_d.at[0]], bd, gd).wait()
            pltpu.make_async_copy(tab.at[idx_s.at[0]], bs, gs).wait()

        def w_start(i, p):
            bd, bs, _, _, wd, ws = sets[p]
            off = pl.multiple_of((w * B + i) * KE, KE)
            pltpu.async_copy(bd, rd_out.at[pl.ds(off, KE), pl.ds(0, D)], wd)
            pltpu.async_copy(bs, rs_out.at[pl.ds(off, KE), pl.ds(0, D)], ws)

        def w_wait(p):
            bd, bs, _, _, wd, ws = sets[p]
            off = pl.multiple_of(w * B * KE, KE)
            pltpu.make_async_copy(bd, rd_out.at[pl.ds(off, KE), pl.ds(0, D)],
                                  wd).wait()
            pltpu.make_async_copy(bs, rs_out.at[pl.ds(off, KE), pl.ds(0, D)],
                                  ws).wait()

        g_start(0, 0)

        def pair(j, carry):
            a = 2 * j

            @pl.when(j > 0)
            def _():
                w_wait(1)

            g_wait(0)
            g_start(a + 1, 1)
            w_start(a, 0)
            g_wait(1)
            w_wait(0)
            w_start(a + 1, 1)
            g_start(a + 2, 0)
            return carry

        lax.fori_loop(0, (B - 1) // 2, pair, 0)
        w_wait(1)
        g_wait(0)
        w_start(B - 1, 0)
        w_wait(0)

    return gather


@functools.cache
def _make_scatter(N, E):
    per_w = E // NW
    B = per_w // KE
    stripe = (N // 16) & ~7
    tail = N - 16 * stripe
    mesh = plsc.VectorSubcoreMesh(core_axis_name="c", subcore_axis_name="s")

    @functools.partial(
        pl.kernel,
        out_type=jax.ShapeDtypeStruct((2, N, D), jnp.float32),
        mesh=mesh,
        compiler_params=pltpu.CompilerParams(use_tc_tiling_on_sc=False),
        scratch_types=[pltpu.VMEM_SHARED((N, D), jnp.float32),
                       pltpu.VMEM((B, KE), jnp.int32)]
        + [pltpu.VMEM((KE, D), jnp.float32)] * 2
        + [pltpu.SemaphoreType.DMA] * 4,
    )
    def scatter(msg, dst3, zeros, out, shared, idx_d, b0, b1,
                l0, l1, s0, s1):
        c = lax.axis_index("c")
        s = lax.axis_index("s")
        w = s * 2 + c
        soff = pl.multiple_of(s * stripe, 8)
        pltpu.sync_copy(zeros.at[pl.ds(soff, stripe)],
                        shared.at[pl.ds(soff, stripe)])
        if tail:
            @pl.when(s == 0)
            def _():
                pltpu.sync_copy(zeros.at[pl.ds(16 * stripe, tail)],
                                shared.at[pl.ds(16 * stripe, tail)])
        pltpu.sync_copy(dst3.at[w], idx_d)
        plsc.subcore_barrier()

        sets = ((b0, l0, s0), (b1, l1, s1))

        def l_start(i, p):
            b, l, _ = sets[p]
            off = pl.multiple_of((w * B + i) * KE, KE)
            pltpu.async_copy(msg.at[pl.ds(off, KE), pl.ds(0, D)], b, l)

        def l_wait(p):
            b, l, _ = sets[p]
            off = pl.multiple_of(w * B * KE, KE)
            pltpu.make_async_copy(msg.at[pl.ds(off, KE), pl.ds(0, D)], b,
                                  l).wait()

        def sc_start(i, p):
            b, _, sm = sets[p]
            pltpu.async_copy(b, shared.at[idx_d.at[i]], sm, add=True)

        def sc_wait(p):
            b, _, sm = sets[p]
            pltpu.make_async_copy(b, shared.at[idx_d.at[0]], sm).wait()

        l_start(0, 0)

        def pair(j, carry):
            a = 2 * j
            l_wait(0)
            sc_start(a, 0)

            @pl.when(j > 0)
            def _():
                sc_wait(1)

            l_start(a + 1, 1)
            l_wait(1)
            sc_start(a + 1, 1)
            sc_wait(0)
            l_start(a + 2, 0)
            return carry

        lax.fori_loop(0, (B - 1) // 2, pair, 0)
        l_wait(0)
        sc_wait(1)
        sc_start(B - 1, 0)
        sc_wait(0)
        plsc.subcore_barrier()
        pltpu.sync_copy(shared.at[pl.ds(soff, stripe)],
                        out.at[c, pl.ds(soff, stripe)])
        if tail:
            @pl.when(s == 0)
            def _():
                pltpu.sync_copy(shared.at[pl.ds(16 * stripe, tail)],
                                out.at[c, pl.ds(16 * stripe, tail)])

    return scatter


def _leaky(v):
    return jnp.where(v > 0, v, 0.01 * v)


def _rsqrt(v):
    r = jax.lax.rsqrt(v)
    return r * (1.5 - 0.5 * v * r * r)


def _sqrt0(v):
    return jnp.where(v > 0, v * _rsqrt(jnp.maximum(v, 1e-30)), 0.0)


def _embed_body(h_ref, w_ref, b_ref, g_ref, bb_ref, o_ref):
    y = jnp.dot(h_ref[...], w_ref[...], preferred_element_type=jnp.float32)
    y = y + b_ref[...]
    mu = jnp.mean(y, axis=0, keepdims=True)
    var = jnp.mean(y * y, axis=0, keepdims=True) - mu * mu
    o_ref[...] = _leaky((y - mu) * _rsqrt(var + 1e-5) * g_ref[...]
                        + bb_ref[...])


def _stats_body(rd_ref, rs_ref, srd_ref, qrd_ref, srs_ref, qrs_ref, dst_ref):
    i = pl.program_id(0)

    @pl.when(i == 0)
    def _():
        srd_ref[...] = jnp.zeros_like(srd_ref)
        qrd_ref[...] = jnp.zeros_like(qrd_ref)
        srs_ref[...] = jnp.zeros_like(srs_ref)
        qrs_ref[...] = jnp.zeros_like(qrs_ref)
        dst_ref[...] = jnp.zeros_like(dst_ref)

    rd = rd_ref[...]
    rs = rs_ref[...]
    diffp = rd[:, 32:48] - rs[:, 32:48]
    dd = jnp.sum(diffp * diffp, axis=1, keepdims=True)
    d = _sqrt0(dd)
    srd_ref[...] += jnp.sum(rd, axis=0, keepdims=True)
    qrd_ref[...] += jnp.sum(rd * rd, axis=0, keepdims=True)
    srs_ref[...] += jnp.sum(rs, axis=0, keepdims=True)
    qrs_ref[...] += jnp.sum(rs * rs, axis=0, keepdims=True)
    dst_ref[...] += jnp.concatenate(
        [jnp.full((1, 8), jnp.sum(d)), jnp.full((1, 8), jnp.sum(dd))], axis=0)


def _edge_body(rd_ref, rs_ref, si_ref, ti_ref, sj_ref, tj_ref, sd_ref,
               td_ref, w1a_ref, w1b_ref, w1d_ref, b1_ref,
               w2_ref, b2_ref, cw1_ref, cb1_ref, cw2_ref, o_ref):
    f32 = jnp.float32
    rd = rd_ref[...]
    rs = rs_ref[...]
    hi_bn = rd[:, :32] * si_ref[...] + ti_ref[...]
    hj_bn = rs[:, :32] * sj_ref[...] + tj_ref[...]
    diffp = rd[:, 32:48] - rs[:, 32:48]
    d = _sqrt0(jnp.sum(diffp * diffp, axis=1, keepdims=True))
    d_bn = d * sd_ref[...] + td_ref[...]
    dq = d_bn.astype(jnp.bfloat16).astype(f32)
    wq = w1d_ref[...].astype(jnp.bfloat16).astype(f32)
    m = jnp.dot(hi_bn, w1a_ref[...], preferred_element_type=f32)
    m = m + jnp.dot(hj_bn, w1b_ref[...], preferred_element_type=f32)
    m = _leaky(m + dq * wq + b1_ref[...])
    m = _leaky(jnp.dot(m, w2_ref[...], preferred_element_type=f32)
               + b2_ref[...])
    cc = _leaky(jnp.dot(m, cw1_ref[...], preferred_element_type=f32)
                + cb1_ref[...])
    c3 = jnp.dot(cc, cw2_ref[...], preferred_element_type=f32)
    o_ref[...] = jnp.concatenate(
        [m, diffp * c3, jnp.zeros((m.shape[0], 80), jnp.float32)], axis=1)


def _node_body(p_ref, h_ref, xp_ref, w1a_ref, w1b_ref, b1_ref, g_ref,
               bb_ref, w2_ref, b2_ref, ho_ref, xo_ref):
    agg = p_ref[0] + p_ref[1]
    m_agg = agg[:, :32]
    xo_ref[...] = xp_ref[...] + agg[:, 32:48]
    h = h_ref[...]
    z = jnp.dot(h, w1a_ref[...], preferred_element_type=jnp.float32)
    z = z + jnp.dot(m_agg, w1b_ref[...], preferred_element_type=jnp.float32)
    z = z + b1_ref[...]
    mu = jnp.mean(z, axis=0, keepdims=True)
    var = jnp.mean(z * z, axis=0, keepdims=True) - mu * mu
    z = _leaky((z - mu) * _rsqrt(var + 1e-5) * g_ref[...] + bb_ref[...])
    ho_ref[...] = jnp.dot(z, w2_ref[...], preferred_element_type=jnp.float32)
    ho_ref[...] += b2_ref[...]


def _out_body(h_ref, w_ref, b_ref, o_ref):
    o_ref[...] = jnp.dot(h_ref[...], w_ref[...],
                         preferred_element_type=jnp.float32) + b_ref[...]


def kernel(h, x, edge_index, params):
    N, IN = h.shape
    E = edge_index.shape[1]
    f32 = jnp.float32

    B = E // NW // KE
    src3 = edge_index[0].astype(jnp.int32).reshape(NW, B, KE)
    dst3 = edge_index[1].astype(jnp.int32).reshape(NW, B, KE)
    xp = jnp.pad(x, ((0, 0), (0, 16 - x.shape[1])))

    r2 = lambda v: v[None, :]

    tc_params = pltpu.CompilerParams(vmem_limit_bytes=100 * 1024 * 1024)
    hh = pl.pallas_call(
        _embed_body,
        compiler_params=tc_params,
        out_shape=jax.ShapeDtypeStruct((N, 32), f32),
    )(h, params['ri_W'].T, r2(params['ri_b']), r2(params['ri_g']),
      r2(params['ri_bb']))

    gather = _make_gather(N, E)
    scatter = _make_scatter(N, E)
    zeros48 = jnp.zeros((N, D), f32)

    KB = 4000
    grid_e = E // KB
    blk = lambda shape: pl.BlockSpec(shape, lambda i: (0, 0))
    eblk = pl.BlockSpec((KB, 128), lambda i: (i, 0))

    for l in range(2):
        p = lambda k: params['l%d_' % l + k]
        tab = jnp.concatenate([hh, xp], axis=1)
        rd, rs = gather(tab, dst3, src3)

        srd, qrd, srs, qrs, dsums = pl.pallas_call(
            _stats_body,
            grid=(grid_e,),
            in_specs=[eblk, eblk],
            out_specs=[blk((1, 128)), blk((1, 128)), blk((1, 128)),
                       blk((1, 128)), blk((2, 8))],
            out_shape=[jax.ShapeDtypeStruct((1, 128), f32)] * 4
            + [jax.ShapeDtypeStruct((2, 8), f32)],
        )(rd, rs)

        mu65 = jnp.concatenate([srd[0, :32], srs[0, :32], dsums[0, :1]]) / E
        ex2 = jnp.concatenate([qrd[0, :32], qrs[0, :32], dsums[1, :1]]) / E
        var65 = ex2 - mu65 * mu65
        s65 = p('ein_g') / jnp.sqrt(var65 + 1e-5)
        t65 = p('ein_b') - mu65 * s65
        W1 = p('e_W1')
        cw2p = jnp.pad(p('c_W2'), ((0, 13), (0, 0))).T

        msg = pl.pallas_call(
            _edge_body,
            grid=(grid_e,),
            in_specs=[eblk, eblk,
                      blk((1, 32)), blk((1, 32)), blk((1, 32)), blk((1, 32)),
                      blk((1, 1)), blk((1, 1)),
                      blk((32, 32)), blk((32, 32)), blk((1, 32)), blk((1, 32)),
                      blk((32, 32)), blk((1, 32)),
                      blk((32, 32)), blk((1, 32)), blk((32, 16))],
            out_specs=eblk,
            out_shape=jax.ShapeDtypeStruct((E, 128), f32),
        )(rd, rs, r2(s65[:32]), r2(t65[:32]), r2(s65[32:64]), r2(t65[32:64]),
          s65[64:].reshape(1, 1), t65[64:].reshape(1, 1),
          W1[:, :32].T, W1[:, 32:64].T, W1[:, 64:65].T, r2(p('e_b1')),
          p('e_W2').T, r2(p('e_b2')), p('c_W1').T, r2(p('c_b1')), cw2p)

        partials = scatter(msg, dst3, zeros48)

        hh, xp = pl.pallas_call(
            _node_body,
            compiler_params=tc_params,
            out_shape=[jax.ShapeDtypeStruct((N, 32), f32),
                       jax.ShapeDtypeStruct((N, 16), f32)],
        )(partials, hh, xp, p('n_W1')[:, :32].T, p('n_W1')[:, 32:].T,
          r2(p('n_b1')), r2(p('n_g')), r2(p('n_bb')), p('n_W2').T,
          r2(p('n_b2')))

    out = pl.pallas_call(
        _out_body,
        compiler_params=tc_params,
        out_shape=jax.ShapeDtypeStruct((N, IN), f32),
    )(hh, params['ro_W'].T, r2(params['ro_b']))

    return jnp.concatenate([out, xp[:, :3]], axis=1)
